# initial kernel scaffold (unmeasured)
import jax
import jax.numpy as jnp
from jax import lax
from jax.experimental import pallas as pl
from jax.experimental.pallas import tpu as pltpu


def kernel(
    x,
):
    def body(*refs):
        pass

    out_shape = jax.ShapeDtypeStruct(..., jnp.float32)
    return pl.pallas_call(body, out_shape=out_shape)(...)


# baseline (device time: 49774 ns/iter reference)
import jax
import jax.numpy as jnp
from jax import lax
from jax.experimental import pallas as pl
from jax.experimental.pallas import tpu as pltpu

N_DEV = 4


def kernel(x):
    m_per, n = x.shape
    c = m_per // N_DEV

    def body(x_ref, out_ref, send_buf, recv_buf, send_sems, recv_sems):
        me = lax.axis_index("i")
        left = (me + N_DEV - 1) % N_DEV
        right = (me + 1) % N_DEV

        barrier_sem = pltpu.get_barrier_semaphore()
        for nbr in (left, right):
            pl.semaphore_signal(
                barrier_sem, inc=1,
                device_id=(nbr,), device_id_type=pl.DeviceIdType.MESH,
            )
        pl.semaphore_wait(barrier_sem, 2)

        send_buf[...] = x_ref[pl.ds(me * c, c), :]
        acc = None
        for s in range(N_DEV - 1):
            rdma = pltpu.make_async_remote_copy(
                src_ref=send_buf,
                dst_ref=recv_buf.at[s],
                send_sem=send_sems.at[s],
                recv_sem=recv_sems.at[s],
                device_id=(right,),
                device_id_type=pl.DeviceIdType.MESH,
            )
            rdma.start()
            rdma.wait()
            idx = (me + 2 * N_DEV - 1 - s) % N_DEV
            acc = recv_buf[s] + x_ref[pl.ds(idx * c, c), :]
            if s < N_DEV - 2:
                send_buf[...] = acc

        own = (me + 1) % N_DEV
        out_ref[pl.ds(own * c, c), :] = acc

        send_buf[...] = acc
        for h in range(N_DEV - 1):
            rdma = pltpu.make_async_remote_copy(
                src_ref=send_buf,
                dst_ref=recv_buf.at[N_DEV - 1 + h],
                send_sem=send_sems.at[N_DEV - 1 + h],
                recv_sem=recv_sems.at[N_DEV - 1 + h],
                device_id=(right,),
                device_id_type=pl.DeviceIdType.MESH,
            )
            rdma.start()
            rdma.wait()
            origin = (me + N_DEV - h) % N_DEV
            out_ref[pl.ds(origin * c, c), :] = recv_buf[N_DEV - 1 + h]
            if h < N_DEV - 2:
                send_buf[...] = recv_buf[N_DEV - 1 + h]

    n_hops = 2 * (N_DEV - 1)
    return pl.pallas_call(
        body,
        out_shape=jax.ShapeDtypeStruct((m_per, n), x.dtype),
        in_specs=[pl.BlockSpec(memory_space=pltpu.VMEM)],
        out_specs=pl.BlockSpec(memory_space=pltpu.VMEM),
        scratch_shapes=[
            pltpu.VMEM((c, n), x.dtype),
            pltpu.VMEM((n_hops, c, n), x.dtype),
            pltpu.SemaphoreType.DMA((n_hops,)),
            pltpu.SemaphoreType.DMA((n_hops,)),
        ],
        compiler_params=pltpu.CompilerParams(collective_id=0),
    )(x)


# device time: 28118 ns/iter; 1.7702x vs baseline; 1.7702x over previous
import jax
import jax.numpy as jnp
from jax import lax
from jax.experimental import pallas as pl
from jax.experimental.pallas import tpu as pltpu

N_DEV = 4


def kernel(x):
    m_per, n = x.shape
    h = m_per // 2
    q = m_per // 4

    def body(x_ref, out_ref, acc_a, acc_b,
             recv_a1, recv_b1, recv_a2, recv_b2, send_sems, recv_sems):
        me = lax.axis_index("i")
        kx = me // 2
        ky = kx ^ (me & 1)
        nbr_y = me ^ 1
        nbr_x = 3 - me

        barrier_sem = pltpu.get_barrier_semaphore()
        for nbr in (nbr_x, nbr_y):
            pl.semaphore_signal(
                barrier_sem, inc=1,
                device_id=(nbr,), device_id_type=pl.DeviceIdType.MESH,
            )
        pl.semaphore_wait(barrier_sem, 2)

        def xchg(src, dst, sem_idx, nbr):
            return pltpu.make_async_remote_copy(
                src_ref=src, dst_ref=dst,
                send_sem=send_sems.at[sem_idx], recv_sem=recv_sems.at[sem_idx],
                device_id=(nbr,), device_id_type=pl.DeviceIdType.MESH,
            )

        a1 = xchg(x_ref.at[pl.ds((1 - kx) * q, q)], recv_a1, 0, nbr_x)
        b1 = xchg(x_ref.at[pl.ds(h + (1 - ky) * q, q)], recv_b1, 1, nbr_y)
        a1.start()
        b1.start()
        a1.wait()
        acc_a[...] = x_ref[pl.ds(kx * q, q), :] + recv_a1[...]

        a2 = xchg(acc_a, recv_a2, 2, nbr_y)
        a2.start()
        b1.wait()
        acc_b[...] = x_ref[pl.ds(h + ky * q, q), :] + recv_b1[...]
        b2 = xchg(acc_b, recv_b2, 3, nbr_x)
        b2.start()

        a_off = kx * q
        b_off = h + ky * q

        a2.wait()
        out_ref[pl.ds(a_off, q), :] = acc_a[...] + recv_a2[...]

        a3 = xchg(out_ref.at[pl.ds(a_off, q)], out_ref.at[pl.ds(a_off, q)],
                  4, nbr_x)
        a3.start()
        b2.wait()
        out_ref[pl.ds(b_off, q), :] = acc_b[...] + recv_b2[...]
        b3 = xchg(out_ref.at[pl.ds(b_off, q)], out_ref.at[pl.ds(b_off, q)],
                  5, nbr_y)
        b3.start()
        a3.wait()
        b3.wait()

    return pl.pallas_call(
        body,
        out_shape=jax.ShapeDtypeStruct((m_per, n), x.dtype),
        in_specs=[pl.BlockSpec(memory_space=pltpu.VMEM)],
        out_specs=pl.BlockSpec(memory_space=pltpu.VMEM),
        scratch_shapes=[
            pltpu.VMEM((q, n), x.dtype),
            pltpu.VMEM((q, n), x.dtype),
            pltpu.VMEM((q, n), x.dtype),
            pltpu.VMEM((q, n), x.dtype),
            pltpu.VMEM((q, n), x.dtype),
            pltpu.VMEM((q, n), x.dtype),
            pltpu.SemaphoreType.DMA((6,)),
            pltpu.SemaphoreType.DMA((6,)),
        ],
        compiler_params=pltpu.CompilerParams(collective_id=0),
    )(x)


# device time: 24706 ns/iter; 2.0147x vs baseline; 1.1381x over previous
import jax
import jax.numpy as jnp
from jax import lax
from jax.experimental import pallas as pl
from jax.experimental.pallas import tpu as pltpu

N_DEV = 4


def kernel(x):
    m_per, n = x.shape
    h = m_per // 2
    q = m_per // 4
    s = q // 2

    def body(x_ref, out_ref, acc_a, acc_b,
             recv_a1, recv_b1, recv_a2, recv_b2, send_sems, recv_sems):
        me = lax.axis_index("i")
        kx = me // 2
        ky = kx ^ (me & 1)
        nbr_y = me ^ 1
        nbr_x = 3 - me

        barrier_sem = pltpu.get_barrier_semaphore()
        for nbr in (nbr_x, nbr_y):
            pl.semaphore_signal(
                barrier_sem, inc=1,
                device_id=(nbr,), device_id_type=pl.DeviceIdType.MESH,
            )
        pl.semaphore_wait(barrier_sem, 2)

        def xchg(src, dst, sem_idx, nbr):
            return pltpu.make_async_remote_copy(
                src_ref=src, dst_ref=dst,
                send_sem=send_sems.at[sem_idx], recv_sem=recv_sems.at[sem_idx],
                device_id=(nbr,), device_id_type=pl.DeviceIdType.MESH,
            )

        a_off = kx * q
        b_off = h + ky * q
        a_send = (1 - kx) * q
        b_send = h + (1 - ky) * q

        a1 = [xchg(x_ref.at[pl.ds(a_send + c * s, s)], recv_a1.at[pl.ds(c * s, s)],
                   0 + c, nbr_x) for c in range(2)]
        b1 = [xchg(x_ref.at[pl.ds(b_send + c * s, s)], recv_b1.at[pl.ds(c * s, s)],
                   2 + c, nbr_y) for c in range(2)]
        a1[0].start()
        b1[0].start()
        a1[1].start()
        b1[1].start()

        a2 = [xchg(acc_a.at[pl.ds(c * s, s)], recv_a2.at[pl.ds(c * s, s)],
                   4 + c, nbr_y) for c in range(2)]
        b2 = [xchg(acc_b.at[pl.ds(c * s, s)], recv_b2.at[pl.ds(c * s, s)],
                   6 + c, nbr_x) for c in range(2)]
        a3 = [xchg(out_ref.at[pl.ds(a_off + c * s, s)],
                   out_ref.at[pl.ds(a_off + c * s, s)], 8 + c, nbr_x)
              for c in range(2)]
        b3 = [xchg(out_ref.at[pl.ds(b_off + c * s, s)],
                   out_ref.at[pl.ds(b_off + c * s, s)], 10 + c, nbr_y)
              for c in range(2)]

        for c in range(2):
            a1[c].wait()
            acc_a[pl.ds(c * s, s), :] = (
                x_ref[pl.ds(a_off + c * s, s), :] + recv_a1[pl.ds(c * s, s), :]
            )
            a2[c].start()
            b1[c].wait()
            acc_b[pl.ds(c * s, s), :] = (
                x_ref[pl.ds(b_off + c * s, s), :] + recv_b1[pl.ds(c * s, s), :]
            )
            b2[c].start()

        for c in range(2):
            a2[c].wait()
            out_ref[pl.ds(a_off + c * s, s), :] = (
                acc_a[pl.ds(c * s, s), :] + recv_a2[pl.ds(c * s, s), :]
            )
            a3[c].start()
            b2[c].wait()
            out_ref[pl.ds(b_off + c * s, s), :] = (
                acc_b[pl.ds(c * s, s), :] + recv_b2[pl.ds(c * s, s), :]
            )
            b3[c].start()

        for c in range(2):
            a3[c].wait()
            b3[c].wait()

    return pl.pallas_call(
        body,
        out_shape=jax.ShapeDtypeStruct((m_per, n), x.dtype),
        in_specs=[pl.BlockSpec(memory_space=pltpu.VMEM)],
        out_specs=pl.BlockSpec(memory_space=pltpu.VMEM),
        scratch_shapes=[
            pltpu.VMEM((q, n), x.dtype),
            pltpu.VMEM((q, n), x.dtype),
            pltpu.VMEM((q, n), x.dtype),
            pltpu.VMEM((q, n), x.dtype),
            pltpu.VMEM((q, n), x.dtype),
            pltpu.VMEM((q, n), x.dtype),
            pltpu.SemaphoreType.DMA((12,)),
            pltpu.SemaphoreType.DMA((12,)),
        ],
        compiler_params=pltpu.CompilerParams(collective_id=0),
    )(x)
